# baseline (speedup 1.0000x reference)
import jax
import jax.numpy as jnp
from jax.experimental import pallas as pl
from jax.experimental.pallas import tpu as pltpu

NODE_DIM = 128
EDGE_DIM = 16
K_PACKED = 288
HIDDEN = 384
DIM_OUT = 128
BLK = 10000


def _mlp_block(xj_ref, xi_ref, ea_ref, ones_ref, w1_ref, w2_ref, b2_ref,
               out_ref):
    xj = xj_ref[:].astype(jnp.bfloat16)
    xi = xi_ref[:].astype(jnp.bfloat16)
    ea = ea_ref[:].astype(jnp.bfloat16)
    x = jnp.concatenate([xj, xi, ea, ones_ref[:]], axis=1)
    h = jnp.dot(x, w1_ref[:], preferred_element_type=jnp.float32)
    h = jnp.maximum(h.astype(jnp.bfloat16), jnp.bfloat16(0.0))
    out_ref[:] = jnp.dot(h, w2_ref[:],
                         preferred_element_type=jnp.float32) + b2_ref[:]


def kernel(x_i, x_j, edge_attr, edge_index, num_nodes, W1, b1, W2, b2):
    del edge_index, num_nodes
    n_edges = x_i.shape[0]
    grid = (n_edges // BLK,)
    w1p = jnp.zeros((K_PACKED, HIDDEN), jnp.bfloat16)
    w1p = w1p.at[:2 * NODE_DIM + EDGE_DIM].set(W1.astype(jnp.bfloat16))
    w1p = w1p.at[2 * NODE_DIM + EDGE_DIM].set(b1.astype(jnp.bfloat16))
    ones = jnp.zeros((BLK, EDGE_DIM), jnp.bfloat16).at[:, 0].set(
        jnp.bfloat16(1.0))
    b2r = b2.reshape(1, DIM_OUT)
    return pl.pallas_call(
        _mlp_block,
        grid=grid,
        in_specs=[
            pl.BlockSpec((BLK, NODE_DIM), lambda i: (i, 0)),
            pl.BlockSpec((BLK, NODE_DIM), lambda i: (i, 0)),
            pl.BlockSpec((BLK, EDGE_DIM), lambda i: (i, 0)),
            pl.BlockSpec((BLK, EDGE_DIM), lambda i: (0, 0)),
            pl.BlockSpec((K_PACKED, HIDDEN), lambda i: (0, 0)),
            pl.BlockSpec((HIDDEN, DIM_OUT), lambda i: (0, 0)),
            pl.BlockSpec((1, DIM_OUT), lambda i: (0, 0)),
        ],
        out_specs=pl.BlockSpec((BLK, DIM_OUT), lambda i: (i, 0)),
        out_shape=jax.ShapeDtypeStruct((n_edges, DIM_OUT), jnp.float32),
        compiler_params=pltpu.CompilerParams(
            dimension_semantics=("parallel",)),
    )(x_j, x_i, edge_attr, ones, w1p, W2.astype(jnp.bfloat16), b2r)


# speedup vs baseline: 1.1078x; 1.1078x over previous
import jax
import jax.numpy as jnp
from jax.experimental import pallas as pl
from jax.experimental.pallas import tpu as pltpu

NODE_DIM = 128
EDGE_DIM = 16
K_PACKED = 288
HIDDEN = 384
DIM_OUT = 128
BLK = 8000


def _mlp_block(xj_ref, xi_ref, ea_ref, ones_ref, w1_ref, w2_ref, b2_ref,
               out_ref):
    xj = xj_ref[:].astype(jnp.bfloat16)
    xi = xi_ref[:].astype(jnp.bfloat16)
    ea = ea_ref[:].astype(jnp.bfloat16)
    x = jnp.concatenate([xj, xi, ea, ones_ref[:]], axis=1)
    h = jnp.dot(x, w1_ref[:], preferred_element_type=jnp.float32)
    h = jnp.maximum(h.astype(jnp.bfloat16), jnp.bfloat16(0.0))
    out_ref[:] = jnp.dot(h, w2_ref[:],
                         preferred_element_type=jnp.float32) + b2_ref[:]


def kernel(x_i, x_j, edge_attr, edge_index, num_nodes, W1, b1, W2, b2):
    del edge_index, num_nodes
    n_edges = x_i.shape[0]
    grid = (n_edges // BLK,)
    w1p = jnp.zeros((K_PACKED, HIDDEN), jnp.bfloat16)
    w1p = w1p.at[:2 * NODE_DIM + EDGE_DIM].set(W1.astype(jnp.bfloat16))
    w1p = w1p.at[2 * NODE_DIM + EDGE_DIM].set(b1.astype(jnp.bfloat16))
    ones = jnp.zeros((BLK, EDGE_DIM), jnp.bfloat16).at[:, 0].set(
        jnp.bfloat16(1.0))
    b2r = b2.reshape(1, DIM_OUT)
    return pl.pallas_call(
        _mlp_block,
        grid=grid,
        in_specs=[
            pl.BlockSpec((BLK, NODE_DIM), lambda i: (i, 0)),
            pl.BlockSpec((BLK, NODE_DIM), lambda i: (i, 0)),
            pl.BlockSpec((BLK, EDGE_DIM), lambda i: (i, 0)),
            pl.BlockSpec((BLK, EDGE_DIM), lambda i: (0, 0)),
            pl.BlockSpec((K_PACKED, HIDDEN), lambda i: (0, 0)),
            pl.BlockSpec((HIDDEN, DIM_OUT), lambda i: (0, 0)),
            pl.BlockSpec((1, DIM_OUT), lambda i: (0, 0)),
        ],
        out_specs=pl.BlockSpec((BLK, DIM_OUT), lambda i: (i, 0)),
        out_shape=jax.ShapeDtypeStruct((n_edges, DIM_OUT), jnp.float32),
        compiler_params=pltpu.CompilerParams(
            dimension_semantics=("parallel",)),
    )(x_j, x_i, edge_attr, ones, w1p, W2.astype(jnp.bfloat16), b2r)


# speedup vs baseline: 1.1233x; 1.0140x over previous
import jax
import jax.numpy as jnp
from jax.experimental import pallas as pl
from jax.experimental.pallas import tpu as pltpu

NODE_DIM = 128
EDGE_DIM = 16
IN_DIM = 272
HIDDEN = 384
DIM_OUT = 128
BLK = 8000


def _mlp_block(xj_ref, xi_ref, ea_ref, w1_ref, b1_ref, w2_ref, b2_ref,
               out_ref):
    xj = xj_ref[:].astype(jnp.bfloat16)
    xi = xi_ref[:].astype(jnp.bfloat16)
    ea = ea_ref[:].astype(jnp.bfloat16)
    x = jnp.concatenate([xj, xi, ea], axis=1)
    h = jnp.dot(x, w1_ref[:].astype(jnp.bfloat16),
                preferred_element_type=jnp.float32)
    h = jnp.maximum((h + b1_ref[:]).astype(jnp.bfloat16), jnp.bfloat16(0.0))
    out_ref[:] = jnp.dot(h, w2_ref[:].astype(jnp.bfloat16),
                         preferred_element_type=jnp.float32) + b2_ref[:]


def kernel(x_i, x_j, edge_attr, edge_index, num_nodes, W1, b1, W2, b2):
    del edge_index, num_nodes
    n_edges = x_i.shape[0]
    grid = (n_edges // BLK,)
    return pl.pallas_call(
        _mlp_block,
        grid=grid,
        in_specs=[
            pl.BlockSpec((BLK, NODE_DIM), lambda i: (i, 0)),
            pl.BlockSpec((BLK, NODE_DIM), lambda i: (i, 0)),
            pl.BlockSpec((BLK, EDGE_DIM), lambda i: (i, 0)),
            pl.BlockSpec((IN_DIM, HIDDEN), lambda i: (0, 0)),
            pl.BlockSpec((1, HIDDEN), lambda i: (0, 0)),
            pl.BlockSpec((HIDDEN, DIM_OUT), lambda i: (0, 0)),
            pl.BlockSpec((1, DIM_OUT), lambda i: (0, 0)),
        ],
        out_specs=pl.BlockSpec((BLK, DIM_OUT), lambda i: (i, 0)),
        out_shape=jax.ShapeDtypeStruct((n_edges, DIM_OUT), jnp.float32),
        compiler_params=pltpu.CompilerParams(
            dimension_semantics=("parallel",)),
    )(x_j, x_i, edge_attr, W1, b1.reshape(1, HIDDEN), W2,
      b2.reshape(1, DIM_OUT))
